# Initial kernel scaffold; baseline (speedup 1.0000x reference)
import jax
import jax.numpy as jnp
from jax import lax
from jax.experimental import pallas as pl
from jax.experimental.pallas import tpu as pltpu

B = 1024
NA = 64
NO = 16
K = 5
H = 128
NN = 2 * NA + NO
NEG = -1e30

_F32 = jnp.float32


def _safe_sqrt(d2):
    safe = jnp.where(d2 > 0, d2, 1.0)
    return jnp.where(d2 > 0, jnp.sqrt(safe), 0.0)


def _gat_kernel(pos_ref, post_ref, vel_ref, velt_ref, *rest):
    params = rest[:-1]
    out_ref = rest[-1]

    pos = pos_ref[0]
    px = pos[:, 0:1]
    py = pos[:, 1:2]
    pxr = post_ref[0, 0:1, :]
    pyr = post_ref[0, 1:2, :]
    vel = vel_ref[0]
    vx = vel[:, 0:1]
    vy = vel[:, 1:2]
    vxr = velt_ref[0, 0:1, :]
    vyr = velt_ref[0, 1:2, :]

    iota_c = lax.broadcasted_iota(jnp.int32, (NN, 1), 0)
    iota_r = lax.broadcasted_iota(jnp.int32, (1, NN), 1)
    iota2 = lax.broadcasted_iota(jnp.int32, (NN, NN), 1)

    dx = px - pxr
    dy = py - pyr
    d2 = dx * dx + dy * dy

    work = d2
    nbr_cols = []
    onehots = []
    for _ in range(K):
        minv = jnp.min(work, axis=1, keepdims=True)
        sel = work == minv
        idx = jnp.min(jnp.where(sel, iota2, NN), axis=1, keepdims=True)
        oh = iota2 == idx
        work = jnp.where(oh, jnp.inf, work)
        nbr_cols.append(idx)
        onehots.append(oh.astype(_F32))

    s5 = iota_c - NA
    in_range = (iota_c >= NA) & (iota_c < NA + NO)
    dup = (nbr_cols[0] == s5)
    for kk in range(1, K):
        dup = dup | (nbr_cols[kk] == s5)
    valid5 = in_range & jnp.logical_not(dup)
    oh5 = ((iota2 == s5) & valid5).astype(_F32)
    onehots.append(oh5)
    nbr_cols.append(jnp.where(valid5, s5, 0))

    lt64_c = iota_c < NA
    lt128_c = iota_c < 2 * NA
    radius_c = jnp.where(lt64_c, 0.05, jnp.where(lt128_c, 0.0, 0.1))
    radius_r = jnp.where(iota_r < NA, 0.05,
                         jnp.where(iota_r < 2 * NA, 0.0, 0.1))
    vnorm = _safe_sqrt(vx * vx + vy * vy)

    attrs = []
    for k in range(K + 1):
        oh = onehots[k]
        sxp = jnp.sum(oh * pxr, axis=1, keepdims=True)
        syp = jnp.sum(oh * pyr, axis=1, keepdims=True)
        sxv = jnp.sum(oh * vxr, axis=1, keepdims=True)
        syv = jnp.sum(oh * vyr, axis=1, keepdims=True)
        srad = jnp.sum(oh * radius_r, axis=1, keepdims=True)
        ddx = sxp - px
        ddy = syp - py
        dist = _safe_sqrt(ddx * ddx + ddy * ddy)
        gap = dist - (srad + radius_c)
        rvx = sxv - vx
        rvy = syv - vy
        invd = 1.0 / jnp.maximum(dist, 1e-6)
        pdx = ddx * invd
        pdy = ddy * invd
        vdot = rvx * pdx + rvy * pdy
        vcrs = rvx * pdy - rvy * pdx
        sidx = nbr_cols[k]
        ag = ((sidx < NA) & (iota_c == sidx + NA)).astype(_F32)
        attrs.append((ag, dist, gap, vdot, vcrs))

    h = None
    n_layers = 3
    for li in range(n_layers):
        Wl, bl, Wr, br, We, att, bo = [params[7 * li + j][...] for j in range(7)]
        if li == 0:
            def lin5(W, b):
                base = jnp.where(lt64_c, W[0:1, :],
                                 jnp.where(lt128_c, W[1:2, :], W[2:3, :]))
                return base + vnorm * W[3:4, :] + radius_c * W[4:5, :] + b
            xl = lin5(Wl, bl)
            xr = lin5(Wr, br)
        else:
            xl = jnp.dot(h, Wl, preferred_element_type=_F32) + bl
            xr = jnp.dot(h, Wr, preferred_element_type=_F32) + br

        logits = []
        xlgs = []
        for k in range(K + 1):
            xlg = jnp.dot(onehots[k], xl, preferred_element_type=_F32)
            ag, dist, gap, vdot, vcrs = attrs[k]
            ew = (ag * We[0:1, :] + dist * We[1:2, :] + gap * We[2:3, :]
                  + vdot * We[3:4, :] + vcrs * We[4:5, :])
            m = xlg + xr + ew
            m = jnp.where(m > 0, m, 0.2 * m)
            logits.append(jnp.sum(m * att, axis=1, keepdims=True))
            xlgs.append(xlg)

        l5 = jnp.where(valid5, logits[K], NEG)
        maxv = l5
        for k in range(K):
            maxv = jnp.maximum(maxv, logits[k])
        exs = [jnp.exp(logits[k] - maxv) for k in range(K)]
        exs.append(jnp.where(valid5, jnp.exp(logits[K] - maxv), 0.0))
        den = exs[0]
        for k in range(1, K + 1):
            den = den + exs[k]
        inv_den = 1.0 / jnp.maximum(den, 1e-16)
        acc = exs[0] * xlgs[0]
        for k in range(1, K + 1):
            acc = acc + exs[k] * xlgs[k]
        h = acc * inv_den + bo
        if li < n_layers - 1:
            h = jnp.maximum(h, 0.0)

    pooled = jnp.sum(jnp.where(lt64_c, h, 0.0))
    out_ref[0, 0] = pooled


def kernel(obstacle_pos, agent_pos, goal_pos, agent_vel, params):
    pos = jnp.concatenate([agent_pos, goal_pos, obstacle_pos], axis=1)
    vel = jnp.concatenate(
        [agent_vel, jnp.zeros((B, NN - NA, 2), dtype=_F32)], axis=1)
    post = jnp.swapaxes(pos, 1, 2)
    velt = jnp.swapaxes(vel, 1, 2)

    flat_params = []
    for (Wl, bl, Wr, br, We, att, bo) in params:
        flat_params += [Wl, bl.reshape(1, -1), Wr, br.reshape(1, -1),
                        We, att.reshape(1, -1), bo.reshape(1, -1)]

    def const_spec(p):
        nd = p.ndim
        return pl.BlockSpec(p.shape, lambda i, _nd=nd: (0,) * _nd)

    grid_spec = pl.GridSpec(
        grid=(B,),
        in_specs=[
            pl.BlockSpec((1, NN, 2), lambda i: (i, 0, 0)),
            pl.BlockSpec((1, 2, NN), lambda i: (i, 0, 0)),
            pl.BlockSpec((1, NN, 2), lambda i: (i, 0, 0)),
            pl.BlockSpec((1, 2, NN), lambda i: (i, 0, 0)),
        ] + [const_spec(p) for p in flat_params],
        out_specs=pl.BlockSpec((1, 1), lambda i: (i, 0)),
    )
    out = pl.pallas_call(
        _gat_kernel,
        grid_spec=grid_spec,
        out_shape=jax.ShapeDtypeStruct((B, 1), _F32),
        compiler_params=pltpu.CompilerParams(
            dimension_semantics=("arbitrary",)),
    )(pos, post, vel, velt, *flat_params)
    return jnp.broadcast_to(out[:, None, :], (B, NA, 1))


# speedup vs baseline: 72.4482x; 72.4482x over previous
import jax
import jax.numpy as jnp
from jax import lax
from jax.experimental import pallas as pl
from jax.experimental.pallas import tpu as pltpu

B = 1024
NA = 64
NO = 16
K = 5
H = 128
NN = 2 * NA + NO
NEG = -1e30

_F32 = jnp.float32


def _safe_sqrt(d2):
    safe = jnp.where(d2 > 0, d2, 1.0)
    return jnp.where(d2 > 0, jnp.sqrt(safe), 0.0)


def _gat_kernel(pos_ref, post_ref, vel_ref, velt_ref, *rest):
    params = rest[:-1]
    out_ref = rest[-1]

    pos = pos_ref[0]
    px = pos[:, 0:1]
    py = pos[:, 1:2]
    pxr = post_ref[0, 0:1, :]
    pyr = post_ref[0, 1:2, :]
    vel = vel_ref[0]
    vx = vel[:, 0:1]
    vy = vel[:, 1:2]
    vxr = velt_ref[0, 0:1, :]
    vyr = velt_ref[0, 1:2, :]

    iota_c = lax.broadcasted_iota(jnp.int32, (NN, 1), 0)
    iota_r = lax.broadcasted_iota(jnp.int32, (1, NN), 1)
    iota2 = lax.broadcasted_iota(jnp.int32, (NN, NN), 1)

    dx = px - pxr
    dy = py - pyr
    d2 = dx * dx + dy * dy

    work = d2
    nbr_cols = []
    onehots = []
    for _ in range(K):
        minv = jnp.min(work, axis=1, keepdims=True)
        sel = work == minv
        idx = jnp.min(jnp.where(sel, iota2, NN), axis=1, keepdims=True)
        oh = iota2 == idx
        work = jnp.where(oh, jnp.inf, work)
        nbr_cols.append(idx)
        onehots.append(oh.astype(_F32))

    s5 = iota_c - NA
    in_range = (iota_c >= NA) & (iota_c < NA + NO)
    dup = (nbr_cols[0] == s5)
    for kk in range(1, K):
        dup = dup | (nbr_cols[kk] == s5)
    valid5 = in_range & jnp.logical_not(dup)
    oh5 = ((iota2 == s5) & valid5).astype(_F32)
    onehots.append(oh5)
    nbr_cols.append(jnp.where(valid5, s5, 0))

    lt64_c = iota_c < NA
    lt128_c = iota_c < 2 * NA
    radius_c = jnp.where(lt64_c, 0.05, jnp.where(lt128_c, 0.0, 0.1))
    radius_r = jnp.where(iota_r < NA, 0.05,
                         jnp.where(iota_r < 2 * NA, 0.0, 0.1))
    vnorm = _safe_sqrt(vx * vx + vy * vy)

    attrs = []
    for k in range(K + 1):
        oh = onehots[k]
        sxp = jnp.sum(oh * pxr, axis=1, keepdims=True)
        syp = jnp.sum(oh * pyr, axis=1, keepdims=True)
        sxv = jnp.sum(oh * vxr, axis=1, keepdims=True)
        syv = jnp.sum(oh * vyr, axis=1, keepdims=True)
        srad = jnp.sum(oh * radius_r, axis=1, keepdims=True)
        ddx = sxp - px
        ddy = syp - py
        dist = _safe_sqrt(ddx * ddx + ddy * ddy)
        gap = dist - (srad + radius_c)
        rvx = sxv - vx
        rvy = syv - vy
        invd = 1.0 / jnp.maximum(dist, 1e-6)
        pdx = ddx * invd
        pdy = ddy * invd
        vdot = rvx * pdx + rvy * pdy
        vcrs = rvx * pdy - rvy * pdx
        sidx = nbr_cols[k]
        ag = ((sidx < NA) & (iota_c == sidx + NA)).astype(_F32)
        attrs.append((ag, dist, gap, vdot, vcrs))

    h = None
    n_layers = 3
    for li in range(n_layers):
        Wl, bl, Wr, br, We, att, bo = [params[7 * li + j][...] for j in range(7)]
        if li == 0:
            def lin5(W, b):
                base = jnp.where(lt64_c, W[0:1, :],
                                 jnp.where(lt128_c, W[1:2, :], W[2:3, :]))
                return base + vnorm * W[3:4, :] + radius_c * W[4:5, :] + b
            xl = lin5(Wl, bl)
            xr = lin5(Wr, br)
        else:
            xl = jnp.dot(h, Wl, preferred_element_type=_F32) + bl
            xr = jnp.dot(h, Wr, preferred_element_type=_F32) + br

        logits = []
        xlgs = []
        for k in range(K + 1):
            xlg = jnp.dot(onehots[k], xl, preferred_element_type=_F32)
            ag, dist, gap, vdot, vcrs = attrs[k]
            ew = (ag * We[0:1, :] + dist * We[1:2, :] + gap * We[2:3, :]
                  + vdot * We[3:4, :] + vcrs * We[4:5, :])
            m = xlg + xr + ew
            m = jnp.where(m > 0, m, 0.2 * m)
            logits.append(jnp.sum(m * att, axis=1, keepdims=True))
            xlgs.append(xlg)

        l5 = jnp.where(valid5, logits[K], NEG)
        maxv = l5
        for k in range(K):
            maxv = jnp.maximum(maxv, logits[k])
        exs = [jnp.exp(logits[k] - maxv) for k in range(K)]
        exs.append(jnp.where(valid5, jnp.exp(logits[K] - maxv), 0.0))
        den = exs[0]
        for k in range(1, K + 1):
            den = den + exs[k]
        inv_den = 1.0 / jnp.maximum(den, 1e-16)
        acc = exs[0] * xlgs[0]
        for k in range(1, K + 1):
            acc = acc + exs[k] * xlgs[k]
        h = acc * inv_den + bo
        if li < n_layers - 1:
            h = jnp.maximum(h, 0.0)

    pooled = jnp.sum(jnp.where(lt64_c, h, 0.0), keepdims=True)
    out_ref[0] = pooled


def kernel(obstacle_pos, agent_pos, goal_pos, agent_vel, params):
    pos = jnp.concatenate([agent_pos, goal_pos, obstacle_pos], axis=1)
    vel = jnp.concatenate(
        [agent_vel, jnp.zeros((B, NN - NA, 2), dtype=_F32)], axis=1)
    post = jnp.swapaxes(pos, 1, 2)
    velt = jnp.swapaxes(vel, 1, 2)

    flat_params = []
    for (Wl, bl, Wr, br, We, att, bo) in params:
        flat_params += [Wl, bl.reshape(1, -1), Wr, br.reshape(1, -1),
                        We, att.reshape(1, -1), bo.reshape(1, -1)]

    def const_spec(p):
        nd = p.ndim
        return pl.BlockSpec(p.shape, lambda i, _nd=nd: (0,) * _nd)

    grid_spec = pl.GridSpec(
        grid=(B,),
        in_specs=[
            pl.BlockSpec((1, NN, 2), lambda i: (i, 0, 0)),
            pl.BlockSpec((1, 2, NN), lambda i: (i, 0, 0)),
            pl.BlockSpec((1, NN, 2), lambda i: (i, 0, 0)),
            pl.BlockSpec((1, 2, NN), lambda i: (i, 0, 0)),
        ] + [const_spec(p) for p in flat_params],
        out_specs=pl.BlockSpec((1, 1, 1), lambda i: (i, 0, 0)),
    )
    out = pl.pallas_call(
        _gat_kernel,
        grid_spec=grid_spec,
        out_shape=jax.ShapeDtypeStruct((B, 1, 1), _F32),
        compiler_params=pltpu.CompilerParams(
            dimension_semantics=("arbitrary",)),
    )(pos, post, vel, velt, *flat_params)
    return jnp.broadcast_to(out, (B, NA, 1))


# speedup vs baseline: 164.2641x; 2.2673x over previous
import jax
import jax.numpy as jnp
from jax import lax
from jax.experimental import pallas as pl
from jax.experimental.pallas import tpu as pltpu

B = 1024
NA = 64
NO = 16
K = 5
H = 128
NN = 2 * NA + NO
NEG = -1e30

_F32 = jnp.float32


def _safe_sqrt(d2):
    safe = jnp.where(d2 > 0, d2, 1.0)
    return jnp.where(d2 > 0, jnp.sqrt(safe), 0.0)


def _tdot(a_t, b):
    return lax.dot_general(a_t, b, (((0,), (0,)), ((), ())),
                           preferred_element_type=_F32)


GPP = 8


def _gat_kernel(pos_ref, featt_ref, *rest):
    params = rest[:-1]
    out_ref = rest[-1]
    for g in range(GPP):
        out_ref[g] = _one_graph(pos_ref[g], featt_ref[g], params)


def _one_graph(pos, featt, params):
    pxr = featt[0:1, :]
    pyr = featt[1:2, :]
    vxr = featt[2:3, :]
    vyr = featt[3:4, :]
    radr = featt[4:5, :]
    px_c = pos[:, 0:1]
    py_c = pos[:, 1:2]

    iota0 = lax.broadcasted_iota(jnp.int32, (NN, NN), 0)
    iota_r = lax.broadcasted_iota(jnp.int32, (1, NN), 1)
    iota_c = lax.broadcasted_iota(jnp.int32, (NN, 1), 0)

    dx = px_c - pxr
    dy = py_c - pyr
    d2 = dx * dx + dy * dy

    work = d2
    idx_rows = []
    onehots = []
    for _ in range(K):
        minv = jnp.min(work, axis=0, keepdims=True)
        sel = work == minv
        idx = jnp.min(jnp.where(sel, iota0, NN), axis=0, keepdims=True)
        oh = iota0 == idx
        work = jnp.where(oh, jnp.inf, work)
        idx_rows.append(idx)
        onehots.append(oh.astype(_F32))

    s5 = iota_r - NA
    in_range = (iota_r >= NA) & (iota_r < NA + NO)
    dup = (idx_rows[0] == s5)
    for kk in range(1, K):
        dup = dup | (idx_rows[kk] == s5)
    valid5 = in_range & jnp.logical_not(dup)
    idx_rows.append(jnp.where(valid5, s5, -1))

    oh5f = ((iota0 == s5) & valid5).astype(_F32)
    ones_c = jnp.ones((NN, 1), dtype=_F32)
    valid5_c = _tdot(oh5f, ones_c) > 0.5

    E6 = 6 * NN
    idx_all = jnp.concatenate(idx_rows, axis=1)
    iota0_all = lax.broadcasted_iota(jnp.int32, (NN, E6), 0)
    oh_all = (iota0_all == idx_all).astype(_F32)
    featt6 = jnp.concatenate([featt] * 6, axis=1)
    iota_d6 = lax.rem(lax.broadcasted_iota(jnp.int32, (1, E6), 1), NN)

    g_all = jnp.dot(featt, oh_all, preferred_element_type=_F32)
    ddx = g_all[0:1, :] - featt6[0:1, :]
    ddy = g_all[1:2, :] - featt6[1:2, :]
    dist = _safe_sqrt(ddx * ddx + ddy * ddy)
    gap = dist - (g_all[4:5, :] + featt6[4:5, :])
    rvx = g_all[2:3, :] - featt6[2:3, :]
    rvy = g_all[3:4, :] - featt6[3:4, :]
    invd = 1.0 / jnp.maximum(dist, 1e-6)
    pdx = ddx * invd
    pdy = ddy * invd
    vdot = rvx * pdx + rvy * pdy
    vcrs = rvx * pdy - rvy * pdx
    ag = ((idx_all < NA) & (iota_d6 == idx_all + NA)).astype(_F32)
    attr8t = jnp.concatenate([ag, dist, gap, vdot, vcrs,
                              jnp.zeros((3, E6), dtype=_F32)], axis=0)

    vnorm = _safe_sqrt(vxr * vxr + vyr * vyr)
    x8t = jnp.concatenate([
        (iota_r < NA).astype(_F32),
        ((iota_r >= NA) & (iota_r < 2 * NA)).astype(_F32),
        (iota_r >= 2 * NA).astype(_F32),
        vnorm,
        radr,
        jnp.zeros((3, NN), dtype=_F32),
    ], axis=0)

    h = None
    n_layers = 3
    for li in range(n_layers):
        Wl, bl, Wr, br, We, att, bo = [params[7 * li + j][...] for j in range(7)]
        if li == 0:
            xl = _tdot(x8t, Wl) + bl
            xr = _tdot(x8t, Wr) + br
        else:
            xl = jnp.dot(h, Wl, preferred_element_type=_F32) + bl
            xr = jnp.dot(h, Wr, preferred_element_type=_F32) + br

        xlg_all = _tdot(oh_all, xl)
        ew_all = _tdot(attr8t, We)
        xr6 = jnp.concatenate([xr] * 6, axis=0)
        m = xlg_all + xr6 + ew_all
        m = jnp.where(m > 0, m, 0.2 * m)
        if li < n_layers - 1:
            lg_all = jnp.dot(m, att, preferred_element_type=_F32)
        else:
            lg_all = m * att
        logits = [lg_all[k * NN:(k + 1) * NN] for k in range(K + 1)]
        xlgs = [xlg_all[k * NN:(k + 1) * NN] for k in range(K + 1)]

        l5 = jnp.where(valid5_c, logits[K], NEG)
        maxv = l5
        for k in range(K):
            maxv = jnp.maximum(maxv, logits[k])
        exs = [jnp.exp(logits[k] - maxv) for k in range(K)]
        exs.append(jnp.where(valid5_c, jnp.exp(logits[K] - maxv), 0.0))
        den = exs[0]
        for k in range(1, K + 1):
            den = den + exs[k]
        inv_den = 1.0 / jnp.maximum(den, 1e-16)
        acc = exs[0] * xlgs[0]
        for k in range(1, K + 1):
            acc = acc + exs[k] * xlgs[k]
        h = acc * inv_den + bo
        if li < n_layers - 1:
            h = jnp.maximum(h, 0.0)

    return jnp.sum(jnp.where(iota_c < NA, h, 0.0), keepdims=True)


def kernel(obstacle_pos, agent_pos, goal_pos, agent_vel, params):
    pos = jnp.concatenate([agent_pos, goal_pos, obstacle_pos], axis=1)
    vel = jnp.concatenate(
        [agent_vel, jnp.zeros((B, NN - NA, 2), dtype=_F32)], axis=1)
    radius = jnp.concatenate([
        jnp.full((NA,), 0.05, dtype=_F32),
        jnp.zeros((NA,), dtype=_F32),
        jnp.full((NO,), 0.1, dtype=_F32),
    ])
    featt = jnp.concatenate([
        jnp.swapaxes(pos, 1, 2),
        jnp.swapaxes(vel, 1, 2),
        jnp.broadcast_to(radius[None, None, :], (B, 1, NN)),
        jnp.zeros((B, 3, NN), dtype=_F32),
    ], axis=1)

    def pad8(W):
        return jnp.concatenate(
            [W, jnp.zeros((3, W.shape[1]), dtype=_F32)], axis=0)

    flat_params = []
    for li, (Wl, bl, Wr, br, We, att, bo) in enumerate(params):
        if li == 0:
            Wl, Wr = pad8(Wl), pad8(Wr)
        flat_params += [Wl, bl.reshape(1, -1), Wr, br.reshape(1, -1),
                        pad8(We), att.reshape(-1, 1), bo.reshape(1, -1)]

    def const_spec(p):
        nd = p.ndim
        return pl.BlockSpec(p.shape, lambda i, _nd=nd: (0,) * _nd)

    grid_spec = pl.GridSpec(
        grid=(B // GPP,),
        in_specs=[
            pl.BlockSpec((GPP, NN, 2), lambda i: (i, 0, 0)),
            pl.BlockSpec((GPP, 8, NN), lambda i: (i, 0, 0)),
        ] + [const_spec(p) for p in flat_params],
        out_specs=pl.BlockSpec((GPP, 1, 1), lambda i: (i, 0, 0)),
    )
    out = pl.pallas_call(
        _gat_kernel,
        grid_spec=grid_spec,
        out_shape=jax.ShapeDtypeStruct((B, 1, 1), _F32),
        compiler_params=pltpu.CompilerParams(
            dimension_semantics=("arbitrary",)),
    )(pos, featt, *flat_params)
    return jnp.broadcast_to(out, (B, NA, 1))


# speedup vs baseline: 168.2925x; 1.0245x over previous
import jax
import jax.numpy as jnp
from jax import lax
from jax.experimental import pallas as pl
from jax.experimental.pallas import tpu as pltpu

B = 1024
NA = 64
NO = 16
K = 5
H = 128
NN = 2 * NA + NO
NEG = -1e30

_F32 = jnp.float32


def _safe_sqrt(d2):
    safe = jnp.where(d2 > 0, d2, 1.0)
    return jnp.where(d2 > 0, jnp.sqrt(safe), 0.0)


def _tdot(a_t, b):
    return lax.dot_general(a_t, b, (((0,), (0,)), ((), ())),
                           preferred_element_type=_F32)


GPP = 16


def _gat_kernel(pos_ref, featt_ref, *rest):
    params = rest[:-1]
    out_ref = rest[-1]
    for g in range(GPP):
        out_ref[g] = _one_graph(pos_ref[g], featt_ref[g], params)


def _one_graph(pos, featt, params):
    pxr = featt[0:1, :]
    pyr = featt[1:2, :]
    vxr = featt[2:3, :]
    vyr = featt[3:4, :]
    radr = featt[4:5, :]
    px_c = pos[:, 0:1]
    py_c = pos[:, 1:2]

    iota0 = lax.broadcasted_iota(jnp.int32, (NN, NN), 0)
    iota_r = lax.broadcasted_iota(jnp.int32, (1, NN), 1)
    iota_c = lax.broadcasted_iota(jnp.int32, (NN, 1), 0)

    dx = px_c - pxr
    dy = py_c - pyr
    d2 = dx * dx + dy * dy

    work = d2
    idx_rows = []
    for _ in range(K):
        minv = jnp.min(work, axis=0, keepdims=True)
        sel = work == minv
        idx = jnp.min(jnp.where(sel, iota0, NN), axis=0, keepdims=True)
        work = jnp.where(iota0 == idx, jnp.inf, work)
        idx_rows.append(idx)

    s5 = iota_r - NA
    in_range = (iota_r >= NA) & (iota_r < NA + NO)
    dup = (idx_rows[0] == s5)
    for kk in range(1, K):
        dup = dup | (idx_rows[kk] == s5)
    valid5 = in_range & jnp.logical_not(dup)
    idx_rows.append(jnp.where(valid5, s5, -1))

    oh5f = ((iota0 == s5) & valid5).astype(_F32)
    ones_c = jnp.ones((NN, 1), dtype=_F32)
    valid5_c = _tdot(oh5f, ones_c) > 0.5

    E6 = 6 * NN
    idx_all = jnp.concatenate(idx_rows, axis=1)
    iota0_all = lax.broadcasted_iota(jnp.int32, (NN, E6), 0)
    oh_all = (iota0_all == idx_all).astype(_F32)
    featt6 = jnp.concatenate([featt] * 6, axis=1)
    iota_d6 = lax.rem(lax.broadcasted_iota(jnp.int32, (1, E6), 1), NN)

    g_all = jnp.dot(featt, oh_all, preferred_element_type=_F32)
    ddx = g_all[0:1, :] - featt6[0:1, :]
    ddy = g_all[1:2, :] - featt6[1:2, :]
    dist = _safe_sqrt(ddx * ddx + ddy * ddy)
    gap = dist - (g_all[4:5, :] + featt6[4:5, :])
    rvx = g_all[2:3, :] - featt6[2:3, :]
    rvy = g_all[3:4, :] - featt6[3:4, :]
    invd = 1.0 / jnp.maximum(dist, 1e-6)
    pdx = ddx * invd
    pdy = ddy * invd
    vdot = rvx * pdx + rvy * pdy
    vcrs = rvx * pdy - rvy * pdx
    ag = ((idx_all < NA) & (iota_d6 == idx_all + NA)).astype(_F32)
    attr8t = jnp.concatenate([ag, dist, gap, vdot, vcrs,
                              jnp.zeros((3, E6), dtype=_F32)], axis=0)

    vnorm = _safe_sqrt(vxr * vxr + vyr * vyr)
    x8t = jnp.concatenate([
        (iota_r < NA).astype(_F32),
        ((iota_r >= NA) & (iota_r < 2 * NA)).astype(_F32),
        (iota_r >= 2 * NA).astype(_F32),
        vnorm,
        radr,
        jnp.zeros((3, NN), dtype=_F32),
    ], axis=0)

    h = None
    n_layers = 3
    for li in range(n_layers):
        Wl, bl, Wr, br, We, att, bo = [params[7 * li + j][...] for j in range(7)]
        if li == 0:
            xl = _tdot(x8t, Wl) + bl
            xr = _tdot(x8t, Wr) + br
        else:
            xl = jnp.dot(h, Wl, preferred_element_type=_F32) + bl
            xr = jnp.dot(h, Wr, preferred_element_type=_F32) + br

        xlg_all = _tdot(oh_all, xl)
        ew_all = _tdot(attr8t, We)
        xr6 = jnp.concatenate([xr] * 6, axis=0)
        m = xlg_all + xr6 + ew_all
        m = jnp.where(m > 0, m, 0.2 * m)
        if li < n_layers - 1:
            lg_all = jnp.dot(m, att, preferred_element_type=_F32)
        else:
            lg_all = m * att
        logits = [lg_all[k * NN:(k + 1) * NN] for k in range(K + 1)]
        xlgs = [xlg_all[k * NN:(k + 1) * NN] for k in range(K + 1)]

        l5 = jnp.where(valid5_c, logits[K], NEG)
        maxv = l5
        for k in range(K):
            maxv = jnp.maximum(maxv, logits[k])
        exs = [jnp.exp(logits[k] - maxv) for k in range(K)]
        exs.append(jnp.where(valid5_c, jnp.exp(logits[K] - maxv), 0.0))
        den = exs[0]
        for k in range(1, K + 1):
            den = den + exs[k]
        inv_den = 1.0 / jnp.maximum(den, 1e-16)
        acc = exs[0] * xlgs[0]
        for k in range(1, K + 1):
            acc = acc + exs[k] * xlgs[k]
        h = acc * inv_den + bo
        if li < n_layers - 1:
            h = jnp.maximum(h, 0.0)

    return jnp.sum(jnp.where(iota_c < NA, h, 0.0), keepdims=True)


def kernel(obstacle_pos, agent_pos, goal_pos, agent_vel, params):
    pos = jnp.concatenate([agent_pos, goal_pos, obstacle_pos], axis=1)
    vel = jnp.concatenate(
        [agent_vel, jnp.zeros((B, NN - NA, 2), dtype=_F32)], axis=1)
    radius = jnp.concatenate([
        jnp.full((NA,), 0.05, dtype=_F32),
        jnp.zeros((NA,), dtype=_F32),
        jnp.full((NO,), 0.1, dtype=_F32),
    ])
    featt = jnp.concatenate([
        jnp.swapaxes(pos, 1, 2),
        jnp.swapaxes(vel, 1, 2),
        jnp.broadcast_to(radius[None, None, :], (B, 1, NN)),
        jnp.zeros((B, 3, NN), dtype=_F32),
    ], axis=1)

    def pad8(W):
        return jnp.concatenate(
            [W, jnp.zeros((3, W.shape[1]), dtype=_F32)], axis=0)

    flat_params = []
    for li, (Wl, bl, Wr, br, We, att, bo) in enumerate(params):
        if li == 0:
            Wl, Wr = pad8(Wl), pad8(Wr)
        flat_params += [Wl, bl.reshape(1, -1), Wr, br.reshape(1, -1),
                        pad8(We), att.reshape(-1, 1), bo.reshape(1, -1)]

    def const_spec(p):
        nd = p.ndim
        return pl.BlockSpec(p.shape, lambda i, _nd=nd: (0,) * _nd)

    grid_spec = pl.GridSpec(
        grid=(B // GPP,),
        in_specs=[
            pl.BlockSpec((GPP, NN, 2), lambda i: (i, 0, 0)),
            pl.BlockSpec((GPP, 8, NN), lambda i: (i, 0, 0)),
        ] + [const_spec(p) for p in flat_params],
        out_specs=pl.BlockSpec((GPP, 1, 1), lambda i: (i, 0, 0)),
    )
    out = pl.pallas_call(
        _gat_kernel,
        grid_spec=grid_spec,
        out_shape=jax.ShapeDtypeStruct((B, 1, 1), _F32),
        compiler_params=pltpu.CompilerParams(
            dimension_semantics=("parallel",)),
    )(pos, featt, *flat_params)
    return jnp.broadcast_to(out, (B, NA, 1))


# speedup vs baseline: 188.8959x; 1.1224x over previous
import jax
import jax.numpy as jnp
from jax import lax
from jax.experimental import pallas as pl
from jax.experimental.pallas import tpu as pltpu

B = 1024
NA = 64
NO = 16
K = 5
H = 128
NN = 2 * NA + NO
NEG = -1e30

_F32 = jnp.float32


def _safe_sqrt(d2):
    safe = jnp.where(d2 > 0, d2, 1.0)
    return jnp.where(d2 > 0, jnp.sqrt(safe), 0.0)


def _tdot(a_t, b):
    return lax.dot_general(a_t, b, (((0,), (0,)), ((), ())),
                           preferred_element_type=_F32)


GPP = 16


def _gat_kernel(pos_ref, featt_ref, *rest):
    params = rest[:-1]
    out_ref = rest[-1]
    for g in range(GPP):
        out_ref[g] = _one_graph(pos_ref[g], featt_ref[g], params)


def _one_graph(pos, featt, params):
    pxr = featt[0:1, :]
    pyr = featt[1:2, :]
    vxr = featt[2:3, :]
    vyr = featt[3:4, :]
    radr = featt[4:5, :]
    px_c = pos[:, 0:1]
    py_c = pos[:, 1:2]

    iota0 = lax.broadcasted_iota(jnp.int32, (NN, NN), 0)
    iota_r = lax.broadcasted_iota(jnp.int32, (1, NN), 1)
    iota_c = lax.broadcasted_iota(jnp.int32, (NN, 1), 0)

    dx = px_c - pxr
    dy = py_c - pyr
    d2 = dx * dx + dy * dy

    work = d2
    idx_rows = []
    for _ in range(K):
        minv = jnp.min(work, axis=0, keepdims=True)
        sel = work == minv
        idx = jnp.min(jnp.where(sel, iota0, NN), axis=0, keepdims=True)
        work = jnp.where(iota0 == idx, jnp.inf, work)
        idx_rows.append(idx)

    s5 = iota_r - NA
    in_range = (iota_r >= NA) & (iota_r < NA + NO)
    dup = (idx_rows[0] == s5)
    for kk in range(1, K):
        dup = dup | (idx_rows[kk] == s5)
    valid5 = in_range & jnp.logical_not(dup)
    v16 = jnp.swapaxes(
        jnp.where(valid5, 1.0, 0.0)[:, NA:NA + NO], 0, 1) > 0.5

    E5 = K * NN
    idx_all = jnp.concatenate(idx_rows, axis=1)
    iota0_all = lax.broadcasted_iota(jnp.int32, (NN, E5), 0)
    oh_all = (iota0_all == idx_all).astype(_F32)
    featt5 = jnp.concatenate([featt] * K, axis=1)
    iota_d5 = lax.rem(lax.broadcasted_iota(jnp.int32, (1, E5), 1), NN)

    g_all = jnp.dot(featt, oh_all, preferred_element_type=_F32)
    ddx = g_all[0:1, :] - featt5[0:1, :]
    ddy = g_all[1:2, :] - featt5[1:2, :]
    dist = _safe_sqrt(ddx * ddx + ddy * ddy)
    gap = dist - (g_all[4:5, :] + featt5[4:5, :])
    rvx = g_all[2:3, :] - featt5[2:3, :]
    rvy = g_all[3:4, :] - featt5[3:4, :]
    invd = 1.0 / jnp.maximum(dist, 1e-6)
    pdx = ddx * invd
    pdy = ddy * invd
    vdot = rvx * pdx + rvy * pdy
    vcrs = rvx * pdy - rvy * pdx
    ag = ((idx_all < NA) & (iota_d5 == idx_all + NA)).astype(_F32)
    attr8t = jnp.concatenate([ag, dist, gap, vdot, vcrs,
                              jnp.zeros((3, E5), dtype=_F32)], axis=0)

    ddx5 = pos[0:NO, 0:1] - pos[NA:NA + NO, 0:1]
    ddy5 = pos[0:NO, 1:2] - pos[NA:NA + NO, 1:2]
    dist5 = _safe_sqrt(ddx5 * ddx5 + ddy5 * ddy5)
    gap5 = dist5 - 0.05
    rvx5 = pos[0:NO, 2:3] - pos[NA:NA + NO, 2:3]
    rvy5 = pos[0:NO, 3:4] - pos[NA:NA + NO, 3:4]
    invd5 = 1.0 / jnp.maximum(dist5, 1e-6)
    pdx5 = ddx5 * invd5
    pdy5 = ddy5 * invd5
    vdot5 = rvx5 * pdx5 + rvy5 * pdy5
    vcrs5 = rvx5 * pdy5 - rvy5 * pdx5
    neg64 = jnp.full((NA, 1), NEG, dtype=_F32)
    zero64 = jnp.zeros((NA, 1), dtype=_F32)

    vnorm = _safe_sqrt(vxr * vxr + vyr * vyr)
    x8t = jnp.concatenate([
        (iota_r < NA).astype(_F32),
        ((iota_r >= NA) & (iota_r < 2 * NA)).astype(_F32),
        (iota_r >= 2 * NA).astype(_F32),
        vnorm,
        radr,
        jnp.zeros((3, NN), dtype=_F32),
    ], axis=0)

    h = None
    n_layers = 3
    for li in range(n_layers):
        Wl, bl, Wr, br, We, att, bo = [params[7 * li + j][...] for j in range(7)]
        if li == 0:
            xl = _tdot(x8t, Wl) + bl
            xr = _tdot(x8t, Wr) + br
        else:
            xl = jnp.dot(h, Wl, preferred_element_type=_F32) + bl
            xr = jnp.dot(h, Wr, preferred_element_type=_F32) + br

        xlg_all = _tdot(oh_all, xl)
        ew_all = _tdot(attr8t, We)
        xr5 = jnp.concatenate([xr] * K, axis=0)
        m = xlg_all + xr5 + ew_all
        m = jnp.maximum(m, 0.2 * m)
        ew5 = (We[0:1, :] + dist5 * We[1:2, :] + gap5 * We[2:3, :]
               + vdot5 * We[3:4, :] + vcrs5 * We[4:5, :])
        m5 = xl[0:NO, :] + xr[NA:NA + NO, :] + ew5
        m5 = jnp.maximum(m5, 0.2 * m5)
        if li < n_layers - 1:
            lg_all = jnp.dot(m, att, preferred_element_type=_F32)
            lg5 = jnp.dot(m5, att, preferred_element_type=_F32)
        else:
            lg_all = m * att
            lg5 = m5 * att
        logits = [lg_all[k * NN:(k + 1) * NN] for k in range(K)]
        xlgs = [xlg_all[k * NN:(k + 1) * NN] for k in range(K)]

        maxv = jnp.concatenate(
            [neg64, jnp.where(v16, lg5, NEG), neg64], axis=0)
        for k in range(K):
            maxv = jnp.maximum(maxv, logits[k])
        exs = [jnp.exp(logits[k] - maxv) for k in range(K)]
        ex5 = jnp.where(v16, jnp.exp(lg5 - maxv[NA:NA + NO]), 0.0)
        ex5_full = jnp.concatenate([zero64, ex5, zero64], axis=0)
        den = ex5_full
        for k in range(K):
            den = den + exs[k]
        inv_den = 1.0 / jnp.maximum(den, 1e-16)
        dout = xl.shape[1]
        xlg5_full = jnp.concatenate([
            jnp.zeros((NA, dout), dtype=_F32),
            xl[0:NO, :],
            jnp.zeros((NA, dout), dtype=_F32),
        ], axis=0)
        acc = ex5_full * xlg5_full
        for k in range(K):
            acc = acc + exs[k] * xlgs[k]
        h = acc * inv_den + bo
        if li < n_layers - 1:
            h = jnp.maximum(h, 0.0)

    return jnp.sum(jnp.where(iota_c < NA, h, 0.0), keepdims=True)


def kernel(obstacle_pos, agent_pos, goal_pos, agent_vel, params):
    pos = jnp.concatenate([agent_pos, goal_pos, obstacle_pos], axis=1)
    vel = jnp.concatenate(
        [agent_vel, jnp.zeros((B, NN - NA, 2), dtype=_F32)], axis=1)
    radius = jnp.concatenate([
        jnp.full((NA,), 0.05, dtype=_F32),
        jnp.zeros((NA,), dtype=_F32),
        jnp.full((NO,), 0.1, dtype=_F32),
    ])
    featt = jnp.concatenate([
        jnp.swapaxes(pos, 1, 2),
        jnp.swapaxes(vel, 1, 2),
        jnp.broadcast_to(radius[None, None, :], (B, 1, NN)),
        jnp.zeros((B, 3, NN), dtype=_F32),
    ], axis=1)
    posvel = jnp.concatenate([pos, vel], axis=2)

    def pad8(W):
        return jnp.concatenate(
            [W, jnp.zeros((3, W.shape[1]), dtype=_F32)], axis=0)

    flat_params = []
    for li, (Wl, bl, Wr, br, We, att, bo) in enumerate(params):
        if li == 0:
            Wl, Wr = pad8(Wl), pad8(Wr)
        flat_params += [Wl, bl.reshape(1, -1), Wr, br.reshape(1, -1),
                        pad8(We), att.reshape(-1, 1), bo.reshape(1, -1)]

    def const_spec(p):
        nd = p.ndim
        return pl.BlockSpec(p.shape, lambda i, _nd=nd: (0,) * _nd)

    grid_spec = pl.GridSpec(
        grid=(B // GPP,),
        in_specs=[
            pl.BlockSpec((GPP, NN, 4), lambda i: (i, 0, 0)),
            pl.BlockSpec((GPP, 8, NN), lambda i: (i, 0, 0)),
        ] + [const_spec(p) for p in flat_params],
        out_specs=pl.BlockSpec((GPP, 1, 1), lambda i: (i, 0, 0)),
    )
    out = pl.pallas_call(
        _gat_kernel,
        grid_spec=grid_spec,
        out_shape=jax.ShapeDtypeStruct((B, 1, 1), _F32),
        compiler_params=pltpu.CompilerParams(
            dimension_semantics=("parallel",)),
    )(posvel, featt, *flat_params)
    return jnp.broadcast_to(out, (B, NA, 1))
